# baseline (speedup 1.0000x reference)
import functools

import jax
import jax.numpy as jnp
from jax import lax
from jax.experimental import pallas as pl


_BLOCK = 2048
_NEG_INF = float("-inf")


def _body(x_ref, w1_ref, b1_ref, w2_ref, b2_ref, val_ref, idx_ref):
    x = x_ref[...]
    w1 = w1_ref[...]
    w2 = w2_ref[...]
    dn = (((1,), (1,)), ((), ()))
    x1 = lax.dot_general(x, w1, dn, preferred_element_type=jnp.float32)
    x1 = x1 + b1_ref[...]
    x2 = jnp.concatenate([x1, x], axis=1)
    x3 = lax.dot_general(x2, w2, dn, preferred_element_type=jnp.float32)
    x3 = x3 + b2_ref[...]

    iota = lax.broadcasted_iota(jnp.int32, x3.shape, 1)
    vals = x3
    out_v = []
    out_i = []
    for _ in range(3):
        m = jnp.max(vals, axis=1, keepdims=True)
        idx = jnp.min(jnp.where(vals == m, iota, 10), axis=1, keepdims=True)
        out_v.append(m)
        out_i.append(idx)
        vals = jnp.where(iota == idx, _NEG_INF, vals)
    val_ref[...] = jnp.concatenate(out_v, axis=1)
    idx_ref[...] = jnp.concatenate(out_i, axis=1)


@functools.partial(jax.jit, static_argnames=())
def kernel(x, W1, b1, W2, b2):
    n = x.shape[0]
    grid = (n // _BLOCK,)
    b1r = b1.reshape(1, 20)
    b2r = b2.reshape(1, 10)
    values, indices = pl.pallas_call(
        _body,
        grid=grid,
        in_specs=[
            pl.BlockSpec((_BLOCK, 10), lambda i: (i, 0)),
            pl.BlockSpec((20, 10), lambda i: (0, 0)),
            pl.BlockSpec((1, 20), lambda i: (0, 0)),
            pl.BlockSpec((10, 30), lambda i: (0, 0)),
            pl.BlockSpec((1, 10), lambda i: (0, 0)),
        ],
        out_specs=[
            pl.BlockSpec((_BLOCK, 3), lambda i: (i, 0)),
            pl.BlockSpec((_BLOCK, 3), lambda i: (i, 0)),
        ],
        out_shape=[
            jax.ShapeDtypeStruct((n, 3), jnp.float32),
            jax.ShapeDtypeStruct((n, 3), jnp.int32),
        ],
    )(x, W1, b1r, W2, b2r)
    return values, indices
